# Initial kernel scaffold; baseline (speedup 1.0000x reference)
import jax
import jax.numpy as jnp
from jax.experimental import pallas as pl


def kernel(x, y, ln_gamma, ln_beta, source_mask, tgt_mask):
    raise NotImplementedError("write your pallas kernel here")


# speedup vs baseline: 1.2928x; 1.2928x over previous
import math
import functools

import jax
import jax.numpy as jnp
from jax.experimental import pallas as pl

N_LAYERS = 2
BUCKET_SIZE = 32
N_HASHES = 4
LN_EPS = 1e-3


def _scores_kernel(bk_ref, qb_ref, kb_ref, o_ref, *, bq, t, d):
    i = pl.program_id(1)
    q = qb_ref[0, pl.ds(i * bq, bq), :]
    dots = jnp.dot(q, kb_ref[0].T, preferred_element_type=jnp.float32)
    dots = dots / math.sqrt(d)

    bq_ids = bk_ref[0, :, pl.ds(i * bq, bq)]
    bk_ids = bk_ref[0]
    same = (bq_ids[0][:, None] == bk_ids[0][None, :])
    for h in range(1, N_HASHES):
        same = same | (bq_ids[h][:, None] == bk_ids[h][None, :])

    rows = jax.lax.broadcasted_iota(jnp.int32, (bq, t), 0) + i * bq
    cols = jax.lax.broadcasted_iota(jnp.int32, (bq, t), 1)
    diag = rows == cols

    scores = jnp.where(same, dots, -1e9)
    o_ref[0] = scores - jnp.where(diag, 1e5, 0.0).astype(jnp.float32)


def _masked_scores(buckets, qk):
    b, t, d = qk.shape
    bq = min(256, t)
    body = functools.partial(_scores_kernel, bq=bq, t=t, d=d)
    kn = qk / (jnp.linalg.norm(qk, axis=-1, keepdims=True) + 1e-9)
    qb = qk.astype(jnp.bfloat16)
    kb = kn.astype(jnp.bfloat16)
    return pl.pallas_call(
        body,
        grid=(b, t // bq),
        in_specs=[
            pl.BlockSpec((1, N_HASHES, t), lambda bb, ii: (bb, 0, 0)),
            pl.BlockSpec((1, t, d), lambda bb, ii: (bb, 0, 0)),
            pl.BlockSpec((1, t, d), lambda bb, ii: (bb, 0, 0)),
        ],
        out_specs=pl.BlockSpec((1, bq, t), lambda bb, ii: (bb, ii, 0)),
        out_shape=jax.ShapeDtypeStruct((b, t, t), jnp.float32),
    )(buckets, qb, kb)


def _bucket_ids(qk, rot):
    rotated = jnp.einsum('btd,dhr->bhtr', qk, rot)
    return jnp.argmax(
        jnp.concatenate([rotated, -rotated], axis=-1), axis=-1
    ).astype(jnp.int32)


def _ln(x, g, b):
    mu = jnp.mean(x, axis=-1, keepdims=True)
    var = jnp.var(x, axis=-1, keepdims=True)
    return (x - mu) / jnp.sqrt(var + LN_EPS) * g + b


def _attn(buckets, qk, v):
    scores = _masked_scores(buckets, qk)
    return jnp.einsum('bij,bjd->bid', jax.nn.softmax(scores, axis=-1), v)


def kernel(x, y, ln_gamma, ln_beta, source_mask, tgt_mask):
    b, t, d = x.shape
    nb = t // BUCKET_SIZE
    base = jax.random.key(1234)
    for l in range(N_LAYERS):
        g = ln_gamma[l]
        bta = ln_beta[l]
        r1 = jax.random.fold_in(base, 2 * l)
        r2 = jax.random.fold_in(base, 2 * l + 1)
        rot1 = jax.random.normal(r1, (d, N_HASHES, nb // 2), dtype=jnp.float32)
        rot2 = jax.random.normal(r2, (d, N_HASHES, nb // 2), dtype=jnp.float32)
        tgt = _ln(_attn(_bucket_ids(y, rot1), y, y), g, bta)
        x = _ln(_attn(_bucket_ids(tgt, rot2), tgt, x), g, bta)
    return (x, y)


# speedup vs baseline: 1.3365x; 1.0337x over previous
import math
import functools

import jax
import jax.numpy as jnp
from jax.experimental import pallas as pl

N_LAYERS = 2
BUCKET_SIZE = 32
N_HASHES = 4
LN_EPS = 1e-3


def _scores_kernel(bk_ref, qb_ref, kb_ref, o_ref, *, bq, t, d):
    i = pl.program_id(1)
    q = qb_ref[0, pl.ds(i * bq, bq), :]
    dots = jnp.dot(q, kb_ref[0].T, preferred_element_type=jnp.float32)
    dots = dots / math.sqrt(d)

    bq_ids = bk_ref[0, :, pl.ds(i * bq, bq)]
    bk_ids = bk_ref[0]
    same = (bq_ids[0][:, None] == bk_ids[0][None, :])
    for h in range(1, N_HASHES):
        same = same | (bq_ids[h][:, None] == bk_ids[h][None, :])

    rows = jax.lax.broadcasted_iota(jnp.int32, (bq, t), 0) + i * bq
    cols = jax.lax.broadcasted_iota(jnp.int32, (bq, t), 1)
    diag = rows == cols

    scores = jnp.where(same, dots, -1e9)
    o_ref[0] = scores - jnp.where(diag, 1e5, 0.0).astype(jnp.float32)


def _masked_scores(buckets, qk):
    b, t, d = qk.shape
    bq = min(256, t)
    body = functools.partial(_scores_kernel, bq=bq, t=t, d=d)
    kn = qk / (jnp.linalg.norm(qk, axis=-1, keepdims=True) + 1e-9)
    qb = qk.astype(jnp.bfloat16)
    kb = kn.astype(jnp.bfloat16)
    return pl.pallas_call(
        body,
        grid=(b, t // bq),
        in_specs=[
            pl.BlockSpec((1, N_HASHES, t), lambda bb, ii: (bb, 0, 0)),
            pl.BlockSpec((1, t, d), lambda bb, ii: (bb, 0, 0)),
            pl.BlockSpec((1, t, d), lambda bb, ii: (bb, 0, 0)),
        ],
        out_specs=pl.BlockSpec((1, bq, t), lambda bb, ii: (bb, ii, 0)),
        out_shape=jax.ShapeDtypeStruct((b, t, t), jnp.float32),
    )(buckets, qb, kb)


def _attn_ln_kernel(bk_ref, qb_ref, kb_ref, vb_ref, g_ref, b_ref, o_ref, *,
                    bq, t, d):
    i = pl.program_id(1)
    q = qb_ref[0, pl.ds(i * bq, bq), :]
    dots = jnp.dot(q, kb_ref[0].T, preferred_element_type=jnp.float32)
    dots = dots / math.sqrt(d)

    bq_ids = bk_ref[0, :, pl.ds(i * bq, bq)]
    bk_ids = bk_ref[0]
    same = (bq_ids[0][:, None] == bk_ids[0][None, :])
    for h in range(1, N_HASHES):
        same = same | (bq_ids[h][:, None] == bk_ids[h][None, :])

    rows = jax.lax.broadcasted_iota(jnp.int32, (bq, t), 0) + i * bq
    cols = jax.lax.broadcasted_iota(jnp.int32, (bq, t), 1)
    diag = rows == cols

    scores = jnp.where(same, dots, -1e9)
    scores = scores - jnp.where(diag, 1e5, 0.0).astype(jnp.float32)

    m = jnp.max(scores, axis=1, keepdims=True)
    p = jnp.exp(scores - m)
    s = jnp.sum(p, axis=1, keepdims=True)
    attn = (p / s).astype(jnp.bfloat16)
    out = jnp.dot(attn, vb_ref[0], preferred_element_type=jnp.float32)

    mu = jnp.mean(out, axis=1, keepdims=True)
    xc = out - mu
    var = jnp.mean(xc * xc, axis=1, keepdims=True)
    o_ref[0] = xc / jnp.sqrt(var + LN_EPS) * g_ref[0] + b_ref[0]


def _attn_ln_fused(buckets, qk, v, g, b):
    bt, t, d = qk.shape
    bq = min(256, t)
    body = functools.partial(_attn_ln_kernel, bq=bq, t=t, d=d)
    kn = qk / (jnp.linalg.norm(qk, axis=-1, keepdims=True) + 1e-9)
    qb = qk.astype(jnp.bfloat16)
    kb = kn.astype(jnp.bfloat16)
    vb = v.astype(jnp.bfloat16)
    g2 = jnp.broadcast_to(g, (bt, d))[:, None, :]
    b2 = jnp.broadcast_to(b, (bt, d))[:, None, :]
    return pl.pallas_call(
        body,
        grid=(bt, t // bq),
        in_specs=[
            pl.BlockSpec((1, N_HASHES, t), lambda bb, ii: (bb, 0, 0)),
            pl.BlockSpec((1, t, d), lambda bb, ii: (bb, 0, 0)),
            pl.BlockSpec((1, t, d), lambda bb, ii: (bb, 0, 0)),
            pl.BlockSpec((1, t, d), lambda bb, ii: (bb, 0, 0)),
            pl.BlockSpec((1, 1, d), lambda bb, ii: (bb, 0, 0)),
            pl.BlockSpec((1, 1, d), lambda bb, ii: (bb, 0, 0)),
        ],
        out_specs=pl.BlockSpec((1, bq, d), lambda bb, ii: (bb, ii, 0)),
        out_shape=jax.ShapeDtypeStruct((bt, t, d), jnp.float32),
    )(buckets, qb, kb, vb, g2, b2)


def _bucket_ids(qk, rot):
    rotated = jnp.einsum('btd,dhr->bhtr', qk, rot)
    return jnp.argmax(
        jnp.concatenate([rotated, -rotated], axis=-1), axis=-1
    ).astype(jnp.int32)


def _ln(x, g, b):
    mu = jnp.mean(x, axis=-1, keepdims=True)
    var = jnp.var(x, axis=-1, keepdims=True)
    return (x - mu) / jnp.sqrt(var + LN_EPS) * g + b


def _attn(buckets, qk, v):
    scores = _masked_scores(buckets, qk)
    return jnp.einsum('bij,bjd->bid', jax.nn.softmax(scores, axis=-1), v)


def kernel(x, y, ln_gamma, ln_beta, source_mask, tgt_mask):
    b, t, d = x.shape
    nb = t // BUCKET_SIZE
    base = jax.random.key(1234)
    for l in range(N_LAYERS):
        g = ln_gamma[l]
        bta = ln_beta[l]
        r1 = jax.random.fold_in(base, 2 * l)
        r2 = jax.random.fold_in(base, 2 * l + 1)
        rot1 = jax.random.normal(r1, (d, N_HASHES, nb // 2), dtype=jnp.float32)
        rot2 = jax.random.normal(r2, (d, N_HASHES, nb // 2), dtype=jnp.float32)
        tgt = _ln(_attn(_bucket_ids(y, rot1), y, y), g, bta)
        x = _attn_ln_fused(_bucket_ids(tgt, rot2), tgt, x, g, bta)
    return (x, y)


# speedup vs baseline: 1.3472x; 1.0080x over previous
import math
import functools

import jax
import jax.numpy as jnp
from jax.experimental import pallas as pl

N_LAYERS = 2
BUCKET_SIZE = 32
N_HASHES = 4
LN_EPS = 1e-3


def _scores_kernel(bk_ref, qb_ref, kb_ref, o_ref, *, bq, t, d):
    i = pl.program_id(1)
    q = qb_ref[0, pl.ds(i * bq, bq), :]
    dots = jnp.dot(q, kb_ref[0].T, preferred_element_type=jnp.float32)
    dots = dots / math.sqrt(d)

    bq_ids = bk_ref[0, :, pl.ds(i * bq, bq)]
    bk_ids = bk_ref[0]
    same = (bq_ids[0][:, None] == bk_ids[0][None, :])
    for h in range(1, N_HASHES):
        same = same | (bq_ids[h][:, None] == bk_ids[h][None, :])

    rows = jax.lax.broadcasted_iota(jnp.int32, (bq, t), 0) + i * bq
    cols = jax.lax.broadcasted_iota(jnp.int32, (bq, t), 1)
    diag = rows == cols

    scores = jnp.where(same, dots, -1e9)
    o_ref[0] = scores - jnp.where(diag, 1e5, 0.0).astype(jnp.float32)


def _masked_scores(buckets, qk):
    b, t, d = qk.shape
    bq = min(512, t)
    body = functools.partial(_scores_kernel, bq=bq, t=t, d=d)
    kn = qk / (jnp.linalg.norm(qk, axis=-1, keepdims=True) + 1e-9)
    qb = qk.astype(jnp.bfloat16)
    kb = kn.astype(jnp.bfloat16)
    return pl.pallas_call(
        body,
        grid=(b, t // bq),
        in_specs=[
            pl.BlockSpec((1, N_HASHES, t), lambda bb, ii: (bb, 0, 0)),
            pl.BlockSpec((1, t, d), lambda bb, ii: (bb, 0, 0)),
            pl.BlockSpec((1, t, d), lambda bb, ii: (bb, 0, 0)),
        ],
        out_specs=pl.BlockSpec((1, bq, t), lambda bb, ii: (bb, ii, 0)),
        out_shape=jax.ShapeDtypeStruct((b, t, t), jnp.float32),
    )(buckets, qb, kb)


def _attn_ln_kernel(bk_ref, qb_ref, kb_ref, vb_ref, g_ref, b_ref, o_ref, *,
                    bq, t, d):
    i = pl.program_id(1)
    q = qb_ref[0, pl.ds(i * bq, bq), :]
    dots = jnp.dot(q, kb_ref[0].T, preferred_element_type=jnp.float32)
    dots = dots / math.sqrt(d)

    bq_ids = bk_ref[0, :, pl.ds(i * bq, bq)]
    bk_ids = bk_ref[0]
    same = (bq_ids[0][:, None] == bk_ids[0][None, :])
    for h in range(1, N_HASHES):
        same = same | (bq_ids[h][:, None] == bk_ids[h][None, :])

    rows = jax.lax.broadcasted_iota(jnp.int32, (bq, t), 0) + i * bq
    cols = jax.lax.broadcasted_iota(jnp.int32, (bq, t), 1)
    diag = rows == cols

    scores = jnp.where(same, dots, -1e9)
    scores = scores - jnp.where(diag, 1e5, 0.0).astype(jnp.float32)

    m = jnp.max(scores, axis=1, keepdims=True)
    p = jnp.exp(scores - m)
    s = jnp.sum(p, axis=1, keepdims=True)
    attn = (p / s).astype(jnp.bfloat16)
    out = jnp.dot(attn, vb_ref[0], preferred_element_type=jnp.float32)

    mu = jnp.mean(out, axis=1, keepdims=True)
    xc = out - mu
    var = jnp.mean(xc * xc, axis=1, keepdims=True)
    o_ref[0] = xc / jnp.sqrt(var + LN_EPS) * g_ref[0] + b_ref[0]


def _attn_ln_fused(buckets, qk, v, g, b):
    bt, t, d = qk.shape
    bq = min(512, t)
    body = functools.partial(_attn_ln_kernel, bq=bq, t=t, d=d)
    kn = qk / (jnp.linalg.norm(qk, axis=-1, keepdims=True) + 1e-9)
    qb = qk.astype(jnp.bfloat16)
    kb = kn.astype(jnp.bfloat16)
    vb = v.astype(jnp.bfloat16)
    g2 = jnp.broadcast_to(g, (bt, d))[:, None, :]
    b2 = jnp.broadcast_to(b, (bt, d))[:, None, :]
    return pl.pallas_call(
        body,
        grid=(bt, t // bq),
        in_specs=[
            pl.BlockSpec((1, N_HASHES, t), lambda bb, ii: (bb, 0, 0)),
            pl.BlockSpec((1, t, d), lambda bb, ii: (bb, 0, 0)),
            pl.BlockSpec((1, t, d), lambda bb, ii: (bb, 0, 0)),
            pl.BlockSpec((1, t, d), lambda bb, ii: (bb, 0, 0)),
            pl.BlockSpec((1, 1, d), lambda bb, ii: (bb, 0, 0)),
            pl.BlockSpec((1, 1, d), lambda bb, ii: (bb, 0, 0)),
        ],
        out_specs=pl.BlockSpec((1, bq, d), lambda bb, ii: (bb, ii, 0)),
        out_shape=jax.ShapeDtypeStruct((bt, t, d), jnp.float32),
    )(buckets, qb, kb, vb, g2, b2)


def _bucket_ids(qk, rot):
    rotated = jnp.einsum('btd,dhr->bhtr', qk, rot)
    return jnp.argmax(
        jnp.concatenate([rotated, -rotated], axis=-1), axis=-1
    ).astype(jnp.int32)


def _ln(x, g, b):
    mu = jnp.mean(x, axis=-1, keepdims=True)
    var = jnp.var(x, axis=-1, keepdims=True)
    return (x - mu) / jnp.sqrt(var + LN_EPS) * g + b


def _attn(buckets, qk, v):
    scores = _masked_scores(buckets, qk)
    return jnp.einsum('bij,bjd->bid', jax.nn.softmax(scores, axis=-1), v)


def kernel(x, y, ln_gamma, ln_beta, source_mask, tgt_mask):
    b, t, d = x.shape
    nb = t // BUCKET_SIZE
    base = jax.random.key(1234)
    for l in range(N_LAYERS):
        g = ln_gamma[l]
        bta = ln_beta[l]
        r1 = jax.random.fold_in(base, 2 * l)
        r2 = jax.random.fold_in(base, 2 * l + 1)
        rot1 = jax.random.normal(r1, (d, N_HASHES, nb // 2), dtype=jnp.float32)
        rot2 = jax.random.normal(r2, (d, N_HASHES, nb // 2), dtype=jnp.float32)
        tgt = _ln(_attn(_bucket_ids(y, rot1), y, y), g, bta)
        x = _attn_ln_fused(_bucket_ids(tgt, rot2), tgt, x, g, bta)
    return (x, y)
